# baseline (speedup 1.0000x reference)
import functools

import jax
import jax.numpy as jnp
from jax import lax
from jax.experimental import pallas as pl
from jax.experimental.pallas import tpu as pltpu
from jax.experimental.pallas import tpu_sc as plsc

VOCAB = 100000
EMBED = 128
BATCH = 16384
NOISE = 20

NCORES = 2
NSUB = 16
NW = NCORES * NSUB
BPW = BATCH // NW
CB = 16
NCHUNK = BPW // CB
LANES = 16
KCH = EMBED // LANES


def _sc_loss_partials(input_words, output_words, noise_flat,
                      embed_in, embed_out):
  mesh = plsc.VectorSubcoreMesh(core_axis_name="c", subcore_axis_name="s")

  @functools.partial(
      pl.kernel,
      out_type=jax.ShapeDtypeStruct((NW * LANES,), jnp.float32),
      mesh=mesh,
      scratch_types=[
          pltpu.VMEM((BPW,), jnp.int32),
          pltpu.VMEM((BPW,), jnp.int32),
          pltpu.VMEM((BPW * NOISE,), jnp.int32),
          pltpu.VMEM((2, CB, EMBED), jnp.float32),
          pltpu.VMEM((2, CB, EMBED), jnp.float32),
          pltpu.VMEM((2, CB * NOISE, EMBED), jnp.float32),
          pltpu.VMEM((LANES,), jnp.float32),
          pltpu.SemaphoreType.DMA,
          pltpu.SemaphoreType.DMA,
      ],
  )
  def k(in_hbm, out_hbm, nz_hbm, ein_hbm, eout_hbm, part_hbm,
        in_idx, out_idx, nz_idx, iv_rows, ov_rows, nv_rows, tot_v,
        sem0, sem1):
    wid = lax.axis_index("s") * NCORES + lax.axis_index("c")
    base = wid * BPW
    lane = lax.broadcasted_iota(jnp.int32, (LANES,), 0)

    pltpu.sync_copy(in_hbm.at[pl.ds(base, BPW)], in_idx)
    pltpu.sync_copy(out_hbm.at[pl.ds(base, BPW)], out_idx)
    pltpu.sync_copy(nz_hbm.at[pl.ds(base * NOISE, BPW * NOISE)], nz_idx)

    def transfers(c, buf, sem):
      o = c * CB
      ts = [
          pltpu.make_async_copy(
              ein_hbm.at[in_idx.at[pl.ds(o, CB)]], iv_rows.at[buf], sem),
          pltpu.make_async_copy(
              eout_hbm.at[out_idx.at[pl.ds(o, CB)]], ov_rows.at[buf], sem),
      ]
      for p in range(0, CB * NOISE, 128):
        n = min(128, CB * NOISE - p)
        ts.append(pltpu.make_async_copy(
            eout_hbm.at[nz_idx.at[pl.ds(o * NOISE + p, n)]],
            nv_rows.at[buf].at[pl.ds(p, n)], sem))
      return ts

    def fire(c, buf, sem):
      for t in transfers(c, buf, sem):
        t.start()

    def drain(c, buf, sem):
      for t in transfers(c, buf, sem):
        t.wait()

    gd = lax.GatherDimensionNumbers(
        offset_dims=(), collapsed_slice_dims=(0,), start_index_map=(0,))

    def xsum(acc):
      for sh in (8, 4, 2, 1):
        perm = lax.gather(
            acc, (lane ^ sh)[:, None], gd, slice_sizes=(1,),
            mode=lax.GatherScatterMode.PROMISE_IN_BOUNDS)
        acc = acc + perm
      return acc

    def logsig(x):
      t = jnp.exp(-jnp.abs(x))
      z = t / (t + 2.0)
      z2 = z * z
      p = 2.0 * z * (1.0 + z2 * (1.0 / 3.0 + z2 * (0.2 + z2 * (
          1.0 / 7.0 + z2 * (1.0 / 9.0)))))
      return jnp.minimum(x, 0.0) - p

    def compute(c, buf, tot):
      def b_body(bl, tot2):
        iv = [iv_rows[buf, bl, pl.ds(LANES * kk, LANES)] for kk in range(KCH)]
        acc = iv[0] * ov_rows[buf, bl, pl.ds(0, LANES)]
        for kk in range(1, KCH):
          acc = acc + iv[kk] * ov_rows[buf, bl, pl.ds(LANES * kk, LANES)]
        tot2 = tot2 + logsig(xsum(acc))
        for n in range(NOISE):
          r = bl * NOISE + n
          nacc = iv[0] * nv_rows[buf, r, pl.ds(0, LANES)]
          for kk in range(1, KCH):
            nacc = nacc + iv[kk] * nv_rows[buf, r, pl.ds(LANES * kk, LANES)]
          tot2 = tot2 + logsig(-xsum(nacc))
        return tot2

      return lax.fori_loop(0, CB, b_body, tot)

    fire(0, 0, sem0)

    def pair_body(i, tot):
      c0 = 2 * i
      fire(c0 + 1, 1, sem1)
      drain(c0, 0, sem0)
      tot = compute(c0, 0, tot)

      @pl.when(c0 + 2 < NCHUNK)
      def _():
        fire(c0 + 2, 0, sem0)

      drain(c0 + 1, 1, sem1)
      tot = compute(c0 + 1, 1, tot)
      return tot

    tot = lax.fori_loop(0, NCHUNK // 2, pair_body,
                        jnp.zeros((LANES,), jnp.float32))
    tot_v[...] = tot
    pltpu.sync_copy(tot_v, part_hbm.at[pl.ds(wid * LANES, LANES)])

  return k(input_words, output_words, noise_flat, embed_in, embed_out)


def kernel(input_words, output_words, noise_words, embed_in, embed_out):
  input_words = input_words.astype(jnp.int32)
  output_words = output_words.astype(jnp.int32)
  noise_flat = noise_words.astype(jnp.int32).reshape(BATCH * NOISE)
  partials = _sc_loss_partials(input_words, output_words, noise_flat,
                               embed_in, embed_out)
  return -(jnp.sum(partials.reshape(NW, LANES)[:, 0]) / BATCH)


# speedup vs baseline: 1.0536x; 1.0536x over previous
import functools

import jax
import jax.numpy as jnp
from jax import lax
from jax.experimental import pallas as pl
from jax.experimental.pallas import tpu as pltpu
from jax.experimental.pallas import tpu_sc as plsc

VOCAB = 100000
EMBED = 128
BATCH = 16384
NOISE = 20

NCORES = 2
NSUB = 16
NW = NCORES * NSUB
BPW = BATCH // NW
CB = 16
NCHUNK = BPW // CB
LANES = 16
KCH = EMBED // LANES


def _sc_loss_partials(input_words, output_words, noise_flat,
                      embed_in, embed_out):
  mesh = plsc.VectorSubcoreMesh(core_axis_name="c", subcore_axis_name="s")

  @functools.partial(
      pl.kernel,
      out_type=jax.ShapeDtypeStruct((NW * LANES,), jnp.float32),
      mesh=mesh,
      scratch_types=[
          pltpu.VMEM((BPW,), jnp.int32),
          pltpu.VMEM((BPW,), jnp.int32),
          pltpu.VMEM((BPW * NOISE,), jnp.int32),
          pltpu.VMEM((2, CB, EMBED), jnp.float32),
          pltpu.VMEM((2, CB, EMBED), jnp.float32),
          pltpu.VMEM((2, CB * NOISE, EMBED), jnp.float32),
          pltpu.VMEM((LANES,), jnp.float32),
          pltpu.SemaphoreType.DMA,
          pltpu.SemaphoreType.DMA,
      ],
  )
  def k(in_hbm, out_hbm, nz_hbm, ein_hbm, eout_hbm, part_hbm,
        in_idx, out_idx, nz_idx, iv_rows, ov_rows, nv_rows, tot_v,
        sem0, sem1):
    wid = lax.axis_index("s") * NCORES + lax.axis_index("c")
    base = wid * BPW
    lane = lax.broadcasted_iota(jnp.int32, (LANES,), 0)

    pltpu.sync_copy(in_hbm.at[pl.ds(base, BPW)], in_idx)
    pltpu.sync_copy(out_hbm.at[pl.ds(base, BPW)], out_idx)
    pltpu.sync_copy(nz_hbm.at[pl.ds(base * NOISE, BPW * NOISE)], nz_idx)

    def transfers(c, buf, sem):
      o = c * CB
      ts = [
          pltpu.make_async_copy(
              ein_hbm.at[in_idx.at[pl.ds(o, CB)]], iv_rows.at[buf], sem),
          pltpu.make_async_copy(
              eout_hbm.at[out_idx.at[pl.ds(o, CB)]], ov_rows.at[buf], sem),
      ]
      for p in range(0, CB * NOISE, 128):
        n = min(128, CB * NOISE - p)
        ts.append(pltpu.make_async_copy(
            eout_hbm.at[nz_idx.at[pl.ds(o * NOISE + p, n)]],
            nv_rows.at[buf].at[pl.ds(p, n)], sem))
      return ts

    def fire(c, buf, sem):
      for t in transfers(c, buf, sem):
        t.start()

    def drain(c, buf, sem):
      for t in transfers(c, buf, sem):
        t.wait()

    gd = lax.GatherDimensionNumbers(
        offset_dims=(), collapsed_slice_dims=(0,), start_index_map=(0,))

    def xsum(acc):
      for sh in (8, 4, 2, 1):
        perm = lax.gather(
            acc, (lane ^ sh)[:, None], gd, slice_sizes=(1,),
            mode=lax.GatherScatterMode.PROMISE_IN_BOUNDS)
        acc = acc + perm
      return acc

    def logsig(x):
      t = jnp.exp(-jnp.abs(x))
      z = t / (t + 2.0)
      z2 = z * z
      p = 2.0 * z * (1.0 + z2 * (1.0 / 3.0 + z2 * (0.2 + z2 * (
          1.0 / 7.0 + z2 * (1.0 / 9.0)))))
      return jnp.minimum(x, 0.0) - p

    def compute(c, buf, tot):
      def b_body(bl, tot2):
        iv = [iv_rows[buf, bl, pl.ds(LANES * kk, LANES)] for kk in range(KCH)]
        acc = iv[0] * ov_rows[buf, bl, pl.ds(0, LANES)]
        for kk in range(1, KCH):
          acc = acc + iv[kk] * ov_rows[buf, bl, pl.ds(LANES * kk, LANES)]
        s = xsum(acc)
        vec_a = jnp.where(lane == 0, s, 0.0)
        vec_b = jnp.zeros((LANES,), jnp.float32)

        for n in range(NOISE):
          r = bl * NOISE + n
          nacc = iv[0] * nv_rows[buf, r, pl.ds(0, LANES)]
          for kk in range(1, KCH):
            nacc = nacc + iv[kk] * nv_rows[buf, r, pl.ds(LANES * kk, LANES)]
          sn = -xsum(nacc)
          j = n + 1
          if j < LANES:
            vec_a = jnp.where(lane == j, sn, vec_a)
          else:
            vec_b = jnp.where(lane == j - LANES, sn, vec_b)
        tot2 = tot2 + logsig(vec_a)
        tot2 = tot2 + jnp.where(lane < NOISE + 1 - LANES, logsig(vec_b), 0.0)
        return tot2

      return lax.fori_loop(0, CB, b_body, tot)

    fire(0, 0, sem0)

    def pair_body(i, tot):
      c0 = 2 * i
      fire(c0 + 1, 1, sem1)
      drain(c0, 0, sem0)
      tot = compute(c0, 0, tot)

      @pl.when(c0 + 2 < NCHUNK)
      def _():
        fire(c0 + 2, 0, sem0)

      drain(c0 + 1, 1, sem1)
      tot = compute(c0 + 1, 1, tot)
      return tot

    tot = lax.fori_loop(0, NCHUNK // 2, pair_body,
                        jnp.zeros((LANES,), jnp.float32))
    tot_v[...] = tot
    pltpu.sync_copy(tot_v, part_hbm.at[pl.ds(wid * LANES, LANES)])

  return k(input_words, output_words, noise_flat, embed_in, embed_out)


def kernel(input_words, output_words, noise_words, embed_in, embed_out):
  input_words = input_words.astype(jnp.int32)
  output_words = output_words.astype(jnp.int32)
  noise_flat = noise_words.astype(jnp.int32).reshape(BATCH * NOISE)
  partials = _sc_loss_partials(input_words, output_words, noise_flat,
                               embed_in, embed_out)
  return -(jnp.sum(partials) / BATCH)
